# baseline (speedup 1.0000x reference)
import functools

import jax
import jax.numpy as jnp
from jax import lax
from jax.experimental import pallas as pl
from jax.experimental.pallas import tpu as pltpu
from jax.experimental.pallas import tpu_sc as plsc

N = 10000
E = 320000
D_IN = 768
DH_P = 256
CHUNK = DH_P // 2
C_P = 48
NS = 16
NC = 2
NR = 10240
NT = 16384
EPT = E // NS
EB1 = 64
EB2 = 1000
RPT = NR // NS

_mesh = plsc.VectorSubcoreMesh(core_axis_name="c", subcore_axis_name="s", num_cores=2, num_subcores=16)


def _mm1_body(x_ref, w_ref, ns_ref, o_ref):
    x = x_ref[...].astype(jnp.bfloat16)
    w = w_ref[0].astype(jnp.bfloat16)
    o_ref[...] = jnp.dot(x, w, preferred_element_type=jnp.float32) * ns_ref[...]


def _stage_a(feature, w1s, ns_col):
    bn = 2048
    return pl.pallas_call(
        _mm1_body,
        grid=(NR // bn, NC),
        in_specs=[
            pl.BlockSpec((bn, D_IN), lambda i, c: (i, 0)),
            pl.BlockSpec((1, D_IN, CHUNK), lambda i, c: (c, 0, 0)),
            pl.BlockSpec((bn, 1), lambda i, c: (i, 0)),
        ],
        out_specs=pl.BlockSpec((bn, CHUNK), lambda i, c: (c * (NR // bn) + i, 0)),
        out_shape=jax.ShapeDtypeStruct((NC * NR, CHUNK), jnp.float32),
    )(feature, w1s, ns_col)


def _deg_body(ei_hbm, out_hbm, acc_v, ebuf, rbuf, dbuf, shared):
    c = lax.axis_index("c")
    s = lax.axis_index("s")
    zero16 = jnp.zeros((16,), jnp.float32)

    def _zacc(i, _):
        acc_v[pl.ds(i * 16, 16)] = zero16
        return 0

    lax.fori_loop(0, NT // 16, _zacc, 0)

    ones16 = jnp.full((16,), 1.0, jnp.float32)
    base = c * E + s * EPT

    def _blk(b, _):
        pltpu.sync_copy(ei_hbm.at[pl.ds(base + b * 2000, 2000)], ebuf)

        def _chunk(k, _):
            idx = ebuf[pl.ds(k * 16, 16)]
            plsc.addupdate_scatter(acc_v, [idx], ones16)
            return 0

        lax.fori_loop(0, 125, _chunk, 0)
        return 0

    lax.fori_loop(0, EPT // 2000, _blk, 0)

    pltpu.sync_copy(acc_v, shared.at[s])
    plsc.subcore_barrier()

    strip = s * (NT // NS)
    for r in range(NS):
        pltpu.sync_copy(shared.at[r, pl.ds(strip, NT // NS)], rbuf.at[r])

    def _red(k, _):
        tot = rbuf[0, pl.ds(k * 16, 16)]
        for r in range(1, NS):
            tot = tot + rbuf[r, pl.ds(k * 16, 16)]
        x = jnp.maximum(tot, 1.0)
        ii = 0x5F3759DF - lax.shift_right_logical(plsc.bitcast(x, jnp.int32), 1)
        y = plsc.bitcast(ii, jnp.float32)
        for _u in range(3):
            y = y * (1.5 - 0.5 * x * y * y)
        dbuf[pl.ds(k * 16, 16)] = y
        return 0

    lax.fori_loop(0, NT // NS // 16, _red, 0)
    pltpu.sync_copy(dbuf, out_hbm.at[pl.ds(c * NT + strip, NT // NS)])


_stage_b = functools.partial(
    pl.kernel,
    out_type=jax.ShapeDtypeStruct((2 * NT,), jnp.float32),
    mesh=_mesh,
    compiler_params=pltpu.CompilerParams(needs_layout_passes=False),
    scratch_types=[
        pltpu.VMEM((NT,), jnp.float32),
        pltpu.VMEM((2000,), jnp.int32),
        pltpu.VMEM((NS, NT // NS), jnp.float32),
        pltpu.VMEM((NT // NS,), jnp.float32),
        pltpu.VMEM_SHARED((NS, NT), jnp.float32),
    ],
)(_deg_body)


EPTP = 20160
NB1 = EPTP // EB1


def _agg1_body(h_hbm, pk_hbm, out_hbm,
               pk0, pk1, pk2, gi0, gi1, gi2, d0, d1, d2, ew0, ew1, ew2,
               r0, r1, r2, acc, g0, g1, g2, s0, s1, s2):
    c = lax.axis_index("c")
    s = lax.axis_index("s")
    pkv = (pk0, pk1, pk2)
    giv = (gi0, gi1, gi2)
    dsv = (d0, d1, d2)
    ewv = (ew0, ew1, ew2)
    rws = (r0, r1, r2)
    gsem = (g0, g1, g2)
    ssem = (s0, s1, s2)

    zero16 = jnp.zeros((16,), jnp.float32)

    def _zr(i, _):
        for j in range(CHUNK // 16):
            r0[i, pl.ds(j * 16, 16)] = zero16
        return 0

    lax.fori_loop(0, EB1, _zr, 0)

    for z in range(RPT // EB1):
        pltpu.sync_copy(r0, acc.at[pl.ds(s * RPT + z * EB1, EB1)])
    plsc.subcore_barrier()

    base = s * EPTP
    cNR = c * NR
    iota = lax.broadcasted_iota(jnp.int32, (16,), 0)

    def _load_unpack(b, slot):
        pltpu.sync_copy(pk_hbm.at[pl.ds(3 * (base + b * EB1), 3 * EB1)], pkv[slot])

        def _pc(k, _):
            r3 = (iota + k * 16) * 3
            giv[slot][pl.ds(k * 16, 16)] = plsc.load_gather(pkv[slot], [r3]) + cNR
            dsv[slot][pl.ds(k * 16, 16)] = plsc.load_gather(pkv[slot], [r3 + 1])
            ewv[slot][pl.ds(k * 16, 16)] = plsc.bitcast(
                plsc.load_gather(pkv[slot], [r3 + 2]), jnp.float32)
            return 0

        lax.fori_loop(0, EB1 // 16, _pc, 0)

    def _fire_gather(slot):
        return pltpu.async_copy(h_hbm.at[giv[slot]], rws[slot], gsem[slot])

    def _wait_gather(slot):
        pltpu.make_async_copy(h_hbm.at[giv[slot]], rws[slot], gsem[slot]).wait()

    def _fire_scatter(slot):
        pltpu.async_copy(rws[slot], acc.at[dsv[slot]], ssem[slot], add=True)

    def _wait_scatter(slot):
        pltpu.make_async_copy(rws[slot], acc.at[dsv[slot]], ssem[slot]).wait()

    def _multiply(slot):
        rv = rws[slot]
        wsrc = ewv[slot]

        def _me(e, _):
            wv = plsc.load_gather(wsrc, [jnp.full((16,), e, jnp.int32)])
            for j in range(CHUNK // 16):
                rv[e, pl.ds(j * 16, 16)] = rv[e, pl.ds(j * 16, 16)] * wv
            return 0

        lax.fori_loop(0, EB1, _me, 0)

    _load_unpack(0, 0)
    _fire_gather(0)
    _load_unpack(1, 1)
    _fire_gather(1)

    def _steady(t, _):
        for u in range(3):
            b = 3 * t + u
            cur = u
            nxt = (u + 2) % 3

            _wait_gather(cur)
            _multiply(cur)

            @pl.when(b + 2 <= NB1 - 1)
            def _prefetch():
                def _do():
                    _wait_scatter(nxt)

                if u == 0:
                    pl.when(t > 0)(_do)
                else:
                    _do()
                _load_unpack(b + 2, nxt)
                _fire_gather(nxt)

            _fire_scatter(cur)

        return 0

    lax.fori_loop(0, NB1 // 3, _steady, 0)

    _wait_scatter(0)
    _wait_scatter(1)
    _wait_scatter(2)
    plsc.subcore_barrier()
    pltpu.sync_copy(acc.at[pl.ds(s * RPT, RPT)],
                    out_hbm.at[c, pl.ds(s * RPT, RPT)])


_stage_c = functools.partial(
    pl.kernel,
    out_type=jax.ShapeDtypeStruct((NC, NR, CHUNK), jnp.float32),
    mesh=_mesh,
    compiler_params=pltpu.CompilerParams(needs_layout_passes=False),
    scratch_types=[
        pltpu.VMEM((3 * EB1,), jnp.int32),
        pltpu.VMEM((3 * EB1,), jnp.int32),
        pltpu.VMEM((3 * EB1,), jnp.int32),
        pltpu.VMEM((EB1,), jnp.int32),
        pltpu.VMEM((EB1,), jnp.int32),
        pltpu.VMEM((EB1,), jnp.int32),
        pltpu.VMEM((EB1,), jnp.int32),
        pltpu.VMEM((EB1,), jnp.int32),
        pltpu.VMEM((EB1,), jnp.int32),
        pltpu.VMEM((EB1,), jnp.float32),
        pltpu.VMEM((EB1,), jnp.float32),
        pltpu.VMEM((EB1,), jnp.float32),
        pltpu.VMEM((EB1, CHUNK), jnp.float32),
        pltpu.VMEM((EB1, CHUNK), jnp.float32),
        pltpu.VMEM((EB1, CHUNK), jnp.float32),
        pltpu.VMEM_SHARED((NR, CHUNK), jnp.float32),
        pltpu.SemaphoreType.DMA,
        pltpu.SemaphoreType.DMA,
        pltpu.SemaphoreType.DMA,
        pltpu.SemaphoreType.DMA,
        pltpu.SemaphoreType.DMA,
        pltpu.SemaphoreType.DMA,
    ],
)(_agg1_body)


def _mm2_body(a_ref, ns_ref, nd_ref, w_ref, b_ref, o_ref):
    nd = nd_ref[...]
    ns = ns_ref[...]
    acc = None
    for ci in range(NC):
        h = jnp.maximum(a_ref[ci] * nd + b_ref[:, ci * CHUNK:(ci + 1) * CHUNK], 0.0)
        h = (h * ns).astype(jnp.bfloat16)
        w = w_ref[ci].astype(jnp.bfloat16)
        y = jnp.dot(h, w, preferred_element_type=jnp.float32)
        acc = y if acc is None else acc + y
    o_ref[...] = acc


def _stage_d(agg1, ns_col, nd_col, w2s, b1p):
    bn = 2048
    return pl.pallas_call(
        _mm2_body,
        grid=(NR // bn,),
        in_specs=[
            pl.BlockSpec((NC, bn, CHUNK), lambda i: (0, i, 0)),
            pl.BlockSpec((bn, 1), lambda i: (i, 0)),
            pl.BlockSpec((bn, 1), lambda i: (i, 0)),
            pl.BlockSpec((NC, CHUNK, C_P), lambda i: (0, 0, 0)),
            pl.BlockSpec((1, DH_P), lambda i: (0, 0)),
        ],
        out_specs=pl.BlockSpec((bn, C_P), lambda i: (i, 0)),
        out_shape=jax.ShapeDtypeStruct((NR, C_P), jnp.float32),
    )(agg1, ns_col, nd_col, w2s, b1p)


def _agg2_body(h_hbm, ei_hbm, ew_hbm, out_hbm,
               src_v, dst_v, ew_v, rows_v, acc, sem):
    c = lax.axis_index("c")
    s = lax.axis_index("s")

    zero16 = jnp.zeros((16,), jnp.float32)

    def _zr(i, _):
        for j in range(C_P // 16):
            rows_v[i, pl.ds(j * 16, 16)] = zero16
        return 0

    lax.fori_loop(0, EB2, _zr, 0)

    pltpu.sync_copy(rows_v.at[pl.ds(0, RPT)], acc.at[pl.ds(s * RPT, RPT)])
    plsc.subcore_barrier()

    ept2 = E // (NC * NS)
    base = c * (E // NC) + s * ept2

    def _blk(b, _):
        off = base + b * EB2
        pltpu.sync_copy(ei_hbm.at[pl.ds(off, EB2)], src_v)
        pltpu.sync_copy(ei_hbm.at[pl.ds(E + off, EB2)], dst_v)
        pltpu.sync_copy(ew_hbm.at[pl.ds(off, EB2)], ew_v)

        pltpu.async_copy(h_hbm.at[src_v], rows_v, sem).wait()

        def _me(e, _):
            wv = plsc.load_gather(ew_v, [jnp.full((16,), e, jnp.int32)])
            for j in range(C_P // 16):
                rows_v[e, pl.ds(j * 16, 16)] = rows_v[e, pl.ds(j * 16, 16)] * wv
            return 0

        lax.fori_loop(0, EB2, _me, 0)

        pltpu.sync_copy(rows_v, acc.at[dst_v], add=True)
        return 0

    lax.fori_loop(0, ept2 // EB2, _blk, 0)

    plsc.subcore_barrier()
    pltpu.sync_copy(acc.at[pl.ds(s * RPT, RPT)],
                    out_hbm.at[c, pl.ds(s * RPT, RPT)])


_stage_e = functools.partial(
    pl.kernel,
    out_type=jax.ShapeDtypeStruct((NC, NR, C_P), jnp.float32),
    mesh=_mesh,
    compiler_params=pltpu.CompilerParams(needs_layout_passes=False,
                                         use_tc_tiling_on_sc=False),
    scratch_types=[
        pltpu.VMEM((EB2,), jnp.int32),
        pltpu.VMEM((EB2,), jnp.int32),
        pltpu.VMEM((EB2,), jnp.float32),
        pltpu.VMEM((EB2, C_P), jnp.float32),
        pltpu.VMEM_SHARED((NR, C_P), jnp.float32),
        pltpu.SemaphoreType.DMA,
    ],
)(_agg2_body)


def _fin_body(p_ref, nd_ref, b_ref, o_ref):
    y = (p_ref[0] + p_ref[1]) * nd_ref[...] + b_ref[...]
    o_ref[...] = y[:, :40]


def _stage_f(parts, nd_col, b2p):
    bn = 2048
    return pl.pallas_call(
        _fin_body,
        grid=(NR // bn,),
        in_specs=[
            pl.BlockSpec((NC, bn, C_P), lambda i: (0, i, 0)),
            pl.BlockSpec((bn, 1), lambda i: (i, 0)),
            pl.BlockSpec((1, C_P), lambda i: (0, 0)),
        ],
        out_specs=pl.BlockSpec((bn, 40), lambda i: (i, 0)),
        out_shape=jax.ShapeDtypeStruct((NR, 40), jnp.float32),
    )(parts, nd_col, b2p)


def kernel(feature, edge_index, edge_weight, W1, b1, W2, b2):
    w1p = jnp.pad(W1, ((0, 0), (0, DH_P - W1.shape[1])))
    w1s = jnp.stack([w1p[:, i * CHUNK:(i + 1) * CHUNK] for i in range(NC)])
    w2p = jnp.pad(W2, ((0, DH_P - W2.shape[0]), (0, C_P - W2.shape[1])))
    w2s = jnp.stack([w2p[i * CHUNK:(i + 1) * CHUNK] for i in range(NC)])
    b1p = jnp.pad(b1, (0, DH_P - b1.shape[0])).reshape(1, DH_P)
    b2p = jnp.pad(b2, (0, C_P - b2.shape[0])).reshape(1, C_P)
    ei_flat = edge_index.reshape(2 * E)
    pk3 = jnp.stack([edge_index[0], edge_index[1],
                     lax.bitcast_convert_type(edge_weight, jnp.int32)],
                    axis=1).reshape(NS, E // NS, 3)
    fake = jnp.broadcast_to(jnp.array([[0, NR - 1, 0]], jnp.int32),
                            (NS, EPTP - E // NS, 3)).reshape(NS, EPTP - E // NS, 3)
    pk = jnp.concatenate([pk3, fake], axis=1).reshape(NS * EPTP * 3)

    nsd = _stage_b(ei_flat)
    ns_col = nsd[:NR].reshape(NR, 1)
    nd_col = nsd[NT:NT + NR].reshape(NR, 1)
    h1 = _stage_a(feature, w1s, ns_col)
    agg1 = _stage_c(h1, pk)
    h2 = _stage_d(agg1, ns_col, nd_col, w2s, b1p)
    parts = _stage_e(h2, ei_flat, edge_weight)
    return _stage_f(parts, nd_col, b2p)[:N]


# speedup vs baseline: 1.0220x; 1.0220x over previous
import functools

import jax
import jax.numpy as jnp
from jax import lax
from jax.experimental import pallas as pl
from jax.experimental.pallas import tpu as pltpu
from jax.experimental.pallas import tpu_sc as plsc

N = 10000
E = 320000
D_IN = 768
DH_P = 256
CHUNK = DH_P // 2
C_P = 48
NS = 16
NC = 2
NR = 10240
NT = 16384
EPT = E // NS
EB1 = 64
EB2 = 1000
RPT = NR // NS

_mesh = plsc.VectorSubcoreMesh(core_axis_name="c", subcore_axis_name="s", num_cores=2, num_subcores=16)


def _mm1_body(x_ref, w_ref, ns_ref, o_ref):
    x = x_ref[...].astype(jnp.bfloat16)
    w = w_ref[0].astype(jnp.bfloat16)
    o_ref[...] = jnp.dot(x, w, preferred_element_type=jnp.float32) * ns_ref[...]


def _stage_a(feature, w1s, ns_col):
    bn = 2048
    return pl.pallas_call(
        _mm1_body,
        grid=(NR // bn, NC),
        in_specs=[
            pl.BlockSpec((bn, D_IN), lambda i, c: (i, 0)),
            pl.BlockSpec((1, D_IN, CHUNK), lambda i, c: (c, 0, 0)),
            pl.BlockSpec((bn, 1), lambda i, c: (i, 0)),
        ],
        out_specs=pl.BlockSpec((bn, CHUNK), lambda i, c: (c * (NR // bn) + i, 0)),
        out_shape=jax.ShapeDtypeStruct((NC * NR, CHUNK), jnp.float32),
    )(feature, w1s, ns_col)


def _deg_body(ei_hbm, out_hbm, acc_v, ebuf, rbuf, dbuf, shared):
    c = lax.axis_index("c")
    s = lax.axis_index("s")
    zero16 = jnp.zeros((16,), jnp.float32)

    def _zacc(i, _):
        acc_v[pl.ds(i * 16, 16)] = zero16
        return 0

    lax.fori_loop(0, NT // 16, _zacc, 0)

    ones16 = jnp.full((16,), 1.0, jnp.float32)
    base = c * E + s * EPT

    def _blk(b, _):
        pltpu.sync_copy(ei_hbm.at[pl.ds(base + b * 2000, 2000)], ebuf)

        def _chunk(k, _):
            idx = ebuf[pl.ds(k * 16, 16)]
            plsc.addupdate_scatter(acc_v, [idx], ones16)
            return 0

        lax.fori_loop(0, 125, _chunk, 0)
        return 0

    lax.fori_loop(0, EPT // 2000, _blk, 0)

    pltpu.sync_copy(acc_v, shared.at[s])
    plsc.subcore_barrier()

    strip = s * (NT // NS)
    for r in range(NS):
        pltpu.sync_copy(shared.at[r, pl.ds(strip, NT // NS)], rbuf.at[r])

    def _red(k, _):
        tot = rbuf[0, pl.ds(k * 16, 16)]
        for r in range(1, NS):
            tot = tot + rbuf[r, pl.ds(k * 16, 16)]
        x = jnp.maximum(tot, 1.0)
        ii = 0x5F3759DF - lax.shift_right_logical(plsc.bitcast(x, jnp.int32), 1)
        y = plsc.bitcast(ii, jnp.float32)
        for _u in range(3):
            y = y * (1.5 - 0.5 * x * y * y)
        dbuf[pl.ds(k * 16, 16)] = y
        return 0

    lax.fori_loop(0, NT // NS // 16, _red, 0)
    pltpu.sync_copy(dbuf, out_hbm.at[pl.ds(c * NT + strip, NT // NS)])


_stage_b = functools.partial(
    pl.kernel,
    out_type=jax.ShapeDtypeStruct((2 * NT,), jnp.float32),
    mesh=_mesh,
    compiler_params=pltpu.CompilerParams(needs_layout_passes=False),
    scratch_types=[
        pltpu.VMEM((NT,), jnp.float32),
        pltpu.VMEM((2000,), jnp.int32),
        pltpu.VMEM((NS, NT // NS), jnp.float32),
        pltpu.VMEM((NT // NS,), jnp.float32),
        pltpu.VMEM_SHARED((NS, NT), jnp.float32),
    ],
)(_deg_body)


EPTP = 20160
NB1 = EPTP // EB1


def _agg1_body(h_hbm, pk_hbm, out_hbm,
               pk0, pk1, pk2, gi0, gi1, gi2, d0, d1, d2, ew0, ew1, ew2,
               r0, r1, r2, acc, g0, g1, g2, s0, s1, s2):
    c = lax.axis_index("c")
    s = lax.axis_index("s")
    pkv = (pk0, pk1, pk2)
    giv = (gi0, gi1, gi2)
    dsv = (d0, d1, d2)
    ewv = (ew0, ew1, ew2)
    rws = (r0, r1, r2)
    gsem = (g0, g1, g2)
    ssem = (s0, s1, s2)

    zero16 = jnp.zeros((16,), jnp.float32)

    def _zr(i, _):
        for j in range(CHUNK // 16):
            r0[i, pl.ds(j * 16, 16)] = zero16
        return 0

    lax.fori_loop(0, EB1, _zr, 0)

    for z in range(RPT // EB1):
        pltpu.sync_copy(r0, acc.at[pl.ds(s * RPT + z * EB1, EB1)])
    plsc.subcore_barrier()

    base = s * EPTP
    cNR = c * NR
    iota = lax.broadcasted_iota(jnp.int32, (16,), 0)

    def _load_unpack(b, slot):
        pltpu.sync_copy(pk_hbm.at[pl.ds(3 * (base + b * EB1), 3 * EB1)], pkv[slot])

        def _pc(k, _):
            r3 = (iota + k * 16) * 3
            giv[slot][pl.ds(k * 16, 16)] = plsc.load_gather(pkv[slot], [r3]) + cNR
            dsv[slot][pl.ds(k * 16, 16)] = plsc.load_gather(pkv[slot], [r3 + 1])
            ewv[slot][pl.ds(k * 16, 16)] = plsc.bitcast(
                plsc.load_gather(pkv[slot], [r3 + 2]), jnp.float32)
            return 0

        lax.fori_loop(0, EB1 // 16, _pc, 0)

    def _fire_gather(slot):
        return pltpu.async_copy(h_hbm.at[giv[slot]], rws[slot], gsem[slot])

    def _wait_gather(slot):
        pltpu.make_async_copy(h_hbm.at[giv[slot]], rws[slot], gsem[slot]).wait()

    def _fire_scatter(slot):
        pltpu.async_copy(rws[slot], acc.at[dsv[slot]], ssem[slot], add=True)

    def _wait_scatter(slot):
        pltpu.make_async_copy(rws[slot], acc.at[dsv[slot]], ssem[slot]).wait()

    def _multiply(slot):
        rv = rws[slot]
        wsrc = ewv[slot]

        def _me(q, _):
            e = q * 4
            for u in range(4):
                wv = plsc.load_gather(wsrc, [jnp.full((16,), e + u, jnp.int32)])
                for j in range(CHUNK // 16):
                    rv[e + u, pl.ds(j * 16, 16)] = rv[e + u, pl.ds(j * 16, 16)] * wv
            return 0

        lax.fori_loop(0, EB1 // 4, _me, 0)

    _load_unpack(0, 0)
    _fire_gather(0)
    _load_unpack(1, 1)
    _fire_gather(1)

    def _steady(t, _):
        for u in range(3):
            b = 3 * t + u
            cur = u
            nxt = (u + 2) % 3

            _wait_gather(cur)
            _multiply(cur)

            @pl.when(b + 2 <= NB1 - 1)
            def _prefetch():
                def _do():
                    _wait_scatter(nxt)

                if u == 0:
                    pl.when(t > 0)(_do)
                else:
                    _do()
                _load_unpack(b + 2, nxt)
                _fire_gather(nxt)

            _fire_scatter(cur)

        return 0

    lax.fori_loop(0, NB1 // 3, _steady, 0)

    _wait_scatter(0)
    _wait_scatter(1)
    _wait_scatter(2)
    plsc.subcore_barrier()
    pltpu.sync_copy(acc.at[pl.ds(s * RPT, RPT)],
                    out_hbm.at[c, pl.ds(s * RPT, RPT)])


_stage_c = functools.partial(
    pl.kernel,
    out_type=jax.ShapeDtypeStruct((NC, NR, CHUNK), jnp.float32),
    mesh=_mesh,
    compiler_params=pltpu.CompilerParams(needs_layout_passes=False),
    scratch_types=[
        pltpu.VMEM((3 * EB1,), jnp.int32),
        pltpu.VMEM((3 * EB1,), jnp.int32),
        pltpu.VMEM((3 * EB1,), jnp.int32),
        pltpu.VMEM((EB1,), jnp.int32),
        pltpu.VMEM((EB1,), jnp.int32),
        pltpu.VMEM((EB1,), jnp.int32),
        pltpu.VMEM((EB1,), jnp.int32),
        pltpu.VMEM((EB1,), jnp.int32),
        pltpu.VMEM((EB1,), jnp.int32),
        pltpu.VMEM((EB1,), jnp.float32),
        pltpu.VMEM((EB1,), jnp.float32),
        pltpu.VMEM((EB1,), jnp.float32),
        pltpu.VMEM((EB1, CHUNK), jnp.float32),
        pltpu.VMEM((EB1, CHUNK), jnp.float32),
        pltpu.VMEM((EB1, CHUNK), jnp.float32),
        pltpu.VMEM_SHARED((NR, CHUNK), jnp.float32),
        pltpu.SemaphoreType.DMA,
        pltpu.SemaphoreType.DMA,
        pltpu.SemaphoreType.DMA,
        pltpu.SemaphoreType.DMA,
        pltpu.SemaphoreType.DMA,
        pltpu.SemaphoreType.DMA,
    ],
)(_agg1_body)


def _mm2_body(a_ref, ns_ref, nd_ref, w_ref, b_ref, o_ref):
    nd = nd_ref[...]
    ns = ns_ref[...]
    acc = None
    for ci in range(NC):
        h = jnp.maximum(a_ref[ci] * nd + b_ref[:, ci * CHUNK:(ci + 1) * CHUNK], 0.0)
        h = (h * ns).astype(jnp.bfloat16)
        w = w_ref[ci].astype(jnp.bfloat16)
        y = jnp.dot(h, w, preferred_element_type=jnp.float32)
        acc = y if acc is None else acc + y
    o_ref[...] = acc


def _stage_d(agg1, ns_col, nd_col, w2s, b1p):
    bn = 2048
    return pl.pallas_call(
        _mm2_body,
        grid=(NR // bn,),
        in_specs=[
            pl.BlockSpec((NC, bn, CHUNK), lambda i: (0, i, 0)),
            pl.BlockSpec((bn, 1), lambda i: (i, 0)),
            pl.BlockSpec((bn, 1), lambda i: (i, 0)),
            pl.BlockSpec((NC, CHUNK, C_P), lambda i: (0, 0, 0)),
            pl.BlockSpec((1, DH_P), lambda i: (0, 0)),
        ],
        out_specs=pl.BlockSpec((bn, C_P), lambda i: (i, 0)),
        out_shape=jax.ShapeDtypeStruct((NR, C_P), jnp.float32),
    )(agg1, ns_col, nd_col, w2s, b1p)


def _agg2_body(h_hbm, ei_hbm, ew_hbm, out_hbm,
               src_v, dst_v, ew_v, rows_v, acc, sem):
    c = lax.axis_index("c")
    s = lax.axis_index("s")

    zero16 = jnp.zeros((16,), jnp.float32)

    def _zr(i, _):
        for j in range(C_P // 16):
            rows_v[i, pl.ds(j * 16, 16)] = zero16
        return 0

    lax.fori_loop(0, EB2, _zr, 0)

    pltpu.sync_copy(rows_v.at[pl.ds(0, RPT)], acc.at[pl.ds(s * RPT, RPT)])
    plsc.subcore_barrier()

    ept2 = E // (NC * NS)
    base = c * (E // NC) + s * ept2

    def _blk(b, _):
        off = base + b * EB2
        pltpu.sync_copy(ei_hbm.at[pl.ds(off, EB2)], src_v)
        pltpu.sync_copy(ei_hbm.at[pl.ds(E + off, EB2)], dst_v)
        pltpu.sync_copy(ew_hbm.at[pl.ds(off, EB2)], ew_v)

        pltpu.async_copy(h_hbm.at[src_v], rows_v, sem).wait()

        def _me(e, _):
            wv = plsc.load_gather(ew_v, [jnp.full((16,), e, jnp.int32)])
            for j in range(C_P // 16):
                rows_v[e, pl.ds(j * 16, 16)] = rows_v[e, pl.ds(j * 16, 16)] * wv
            return 0

        lax.fori_loop(0, EB2, _me, 0)

        pltpu.sync_copy(rows_v, acc.at[dst_v], add=True)
        return 0

    lax.fori_loop(0, ept2 // EB2, _blk, 0)

    plsc.subcore_barrier()
    pltpu.sync_copy(acc.at[pl.ds(s * RPT, RPT)],
                    out_hbm.at[c, pl.ds(s * RPT, RPT)])


_stage_e = functools.partial(
    pl.kernel,
    out_type=jax.ShapeDtypeStruct((NC, NR, C_P), jnp.float32),
    mesh=_mesh,
    compiler_params=pltpu.CompilerParams(needs_layout_passes=False,
                                         use_tc_tiling_on_sc=False),
    scratch_types=[
        pltpu.VMEM((EB2,), jnp.int32),
        pltpu.VMEM((EB2,), jnp.int32),
        pltpu.VMEM((EB2,), jnp.float32),
        pltpu.VMEM((EB2, C_P), jnp.float32),
        pltpu.VMEM_SHARED((NR, C_P), jnp.float32),
        pltpu.SemaphoreType.DMA,
    ],
)(_agg2_body)


def _fin_body(p_ref, nd_ref, b_ref, o_ref):
    y = (p_ref[0] + p_ref[1]) * nd_ref[...] + b_ref[...]
    o_ref[...] = y[:, :40]


def _stage_f(parts, nd_col, b2p):
    bn = 2048
    return pl.pallas_call(
        _fin_body,
        grid=(NR // bn,),
        in_specs=[
            pl.BlockSpec((NC, bn, C_P), lambda i: (0, i, 0)),
            pl.BlockSpec((bn, 1), lambda i: (i, 0)),
            pl.BlockSpec((1, C_P), lambda i: (0, 0)),
        ],
        out_specs=pl.BlockSpec((bn, 40), lambda i: (i, 0)),
        out_shape=jax.ShapeDtypeStruct((NR, 40), jnp.float32),
    )(parts, nd_col, b2p)


def kernel(feature, edge_index, edge_weight, W1, b1, W2, b2):
    w1p = jnp.pad(W1, ((0, 0), (0, DH_P - W1.shape[1])))
    w1s = jnp.stack([w1p[:, i * CHUNK:(i + 1) * CHUNK] for i in range(NC)])
    w2p = jnp.pad(W2, ((0, DH_P - W2.shape[0]), (0, C_P - W2.shape[1])))
    w2s = jnp.stack([w2p[i * CHUNK:(i + 1) * CHUNK] for i in range(NC)])
    b1p = jnp.pad(b1, (0, DH_P - b1.shape[0])).reshape(1, DH_P)
    b2p = jnp.pad(b2, (0, C_P - b2.shape[0])).reshape(1, C_P)
    ei_flat = edge_index.reshape(2 * E)
    pk3 = jnp.stack([edge_index[0], edge_index[1],
                     lax.bitcast_convert_type(edge_weight, jnp.int32)],
                    axis=1).reshape(NS, E // NS, 3)
    fake = jnp.broadcast_to(jnp.array([[0, NR - 1, 0]], jnp.int32),
                            (NS, EPTP - E // NS, 3)).reshape(NS, EPTP - E // NS, 3)
    pk = jnp.concatenate([pk3, fake], axis=1).reshape(NS * EPTP * 3)

    nsd = _stage_b(ei_flat)
    ns_col = nsd[:NR].reshape(NR, 1)
    nd_col = nsd[NT:NT + NR].reshape(NR, 1)
    h1 = _stage_a(feature, w1s, ns_col)
    agg1 = _stage_c(h1, pk)
    h2 = _stage_d(agg1, ns_col, nd_col, w2s, b1p)
    parts = _stage_e(h2, ei_flat, edge_weight)
    return _stage_f(parts, nd_col, b2p)[:N]


# speedup vs baseline: 1.1117x; 1.0877x over previous
import functools

import jax
import jax.numpy as jnp
from jax import lax
from jax.experimental import pallas as pl
from jax.experimental.pallas import tpu as pltpu
from jax.experimental.pallas import tpu_sc as plsc

N = 10000
E = 320000
D_IN = 768
DH_P = 224
CHUNK = DH_P // 2
C_P = 48
NS = 16
NC = 2
NR = 10240
NT = 16384
EPT = E // NS
EB1 = 160
EB2 = 1000
RPT = NR // NS

_mesh = plsc.VectorSubcoreMesh(core_axis_name="c", subcore_axis_name="s", num_cores=2, num_subcores=16)


def _mm1_body(x_ref, w_ref, ns_ref, o_ref):
    x = x_ref[...].astype(jnp.bfloat16)
    w = w_ref[0].astype(jnp.bfloat16)
    o_ref[...] = jnp.dot(x, w, preferred_element_type=jnp.float32) * ns_ref[...]


def _stage_a(feature, w1s, ns_col):
    bn = 2048
    return pl.pallas_call(
        _mm1_body,
        grid=(NR // bn, NC),
        in_specs=[
            pl.BlockSpec((bn, D_IN), lambda i, c: (i, 0)),
            pl.BlockSpec((1, D_IN, CHUNK), lambda i, c: (c, 0, 0)),
            pl.BlockSpec((bn, 1), lambda i, c: (i, 0)),
        ],
        out_specs=pl.BlockSpec((bn, CHUNK), lambda i, c: (c * (NR // bn) + i, 0)),
        out_shape=jax.ShapeDtypeStruct((NC * NR, CHUNK), jnp.float32),
    )(feature, w1s, ns_col)


def _deg_body(ei_hbm, out_hbm, acc_v, ebuf, rbuf, dbuf, shared):
    c = lax.axis_index("c")
    s = lax.axis_index("s")
    zero16 = jnp.zeros((16,), jnp.float32)

    def _zacc(i, _):
        acc_v[pl.ds(i * 16, 16)] = zero16
        return 0

    lax.fori_loop(0, NT // 16, _zacc, 0)

    ones16 = jnp.full((16,), 1.0, jnp.float32)
    base = c * E + s * EPT

    def _blk(b, _):
        pltpu.sync_copy(ei_hbm.at[pl.ds(base + b * 2000, 2000)], ebuf)

        def _chunk(k, _):
            idx = ebuf[pl.ds(k * 16, 16)]
            plsc.addupdate_scatter(acc_v, [idx], ones16)
            return 0

        lax.fori_loop(0, 125, _chunk, 0)
        return 0

    lax.fori_loop(0, EPT // 2000, _blk, 0)

    pltpu.sync_copy(acc_v, shared.at[s])
    plsc.subcore_barrier()

    strip = s * (NT // NS)
    for r in range(NS):
        pltpu.sync_copy(shared.at[r, pl.ds(strip, NT // NS)], rbuf.at[r])

    def _red(k, _):
        tot = rbuf[0, pl.ds(k * 16, 16)]
        for r in range(1, NS):
            tot = tot + rbuf[r, pl.ds(k * 16, 16)]
        x = jnp.maximum(tot, 1.0)
        ii = 0x5F3759DF - lax.shift_right_logical(plsc.bitcast(x, jnp.int32), 1)
        y = plsc.bitcast(ii, jnp.float32)
        for _u in range(3):
            y = y * (1.5 - 0.5 * x * y * y)
        dbuf[pl.ds(k * 16, 16)] = y
        return 0

    lax.fori_loop(0, NT // NS // 16, _red, 0)
    pltpu.sync_copy(dbuf, out_hbm.at[pl.ds(c * NT + strip, NT // NS)])


_stage_b = functools.partial(
    pl.kernel,
    out_type=jax.ShapeDtypeStruct((2 * NT,), jnp.float32),
    mesh=_mesh,
    compiler_params=pltpu.CompilerParams(needs_layout_passes=False),
    scratch_types=[
        pltpu.VMEM((NT,), jnp.float32),
        pltpu.VMEM((2000,), jnp.int32),
        pltpu.VMEM((NS, NT // NS), jnp.float32),
        pltpu.VMEM((NT // NS,), jnp.float32),
        pltpu.VMEM_SHARED((NS, NT), jnp.float32),
    ],
)(_deg_body)


EPTP = 20160
NB1 = EPTP // EB1


def _agg1_body(h_hbm, pk_hbm, out_hbm,
               pk0, pk1, pk2, gi0, gi1, gi2, d0, d1, d2, ew0, ew1, ew2,
               r0, r1, r2, acc, g0, g1, g2, s0, s1, s2):
    c = lax.axis_index("c")
    s = lax.axis_index("s")
    pkv = (pk0, pk1, pk2)
    giv = (gi0, gi1, gi2)
    dsv = (d0, d1, d2)
    ewv = (ew0, ew1, ew2)
    rws = (r0, r1, r2)
    gsem = (g0, g1, g2)
    ssem = (s0, s1, s2)

    zero16 = jnp.zeros((16,), jnp.float32)

    def _zr(i, _):
        for j in range(CHUNK // 16):
            r0[i, pl.ds(j * 16, 16)] = zero16
        return 0

    lax.fori_loop(0, EB1, _zr, 0)

    for z in range(RPT // EB1):
        pltpu.sync_copy(r0, acc.at[pl.ds(s * RPT + z * EB1, EB1)])
    plsc.subcore_barrier()

    base = s * EPTP
    cNR = c * NR
    iota = lax.broadcasted_iota(jnp.int32, (16,), 0)

    def _load_unpack(b, slot):
        pltpu.sync_copy(pk_hbm.at[pl.ds(3 * (base + b * EB1), 3 * EB1)], pkv[slot])

        def _pc(k, _):
            r3 = (iota + k * 16) * 3
            giv[slot][pl.ds(k * 16, 16)] = plsc.load_gather(pkv[slot], [r3]) + cNR
            dsv[slot][pl.ds(k * 16, 16)] = plsc.load_gather(pkv[slot], [r3 + 1])
            ewv[slot][pl.ds(k * 16, 16)] = plsc.bitcast(
                plsc.load_gather(pkv[slot], [r3 + 2]), jnp.float32)
            return 0

        lax.fori_loop(0, EB1 // 16, _pc, 0)

    def _fire_gather(slot):
        return pltpu.async_copy(h_hbm.at[giv[slot]], rws[slot], gsem[slot])

    def _wait_gather(slot):
        pltpu.make_async_copy(h_hbm.at[giv[slot]], rws[slot], gsem[slot]).wait()

    def _fire_scatter(slot):
        pltpu.async_copy(rws[slot], acc.at[dsv[slot]], ssem[slot], add=True)

    def _wait_scatter(slot):
        pltpu.make_async_copy(rws[slot], acc.at[dsv[slot]], ssem[slot]).wait()

    def _multiply(slot):
        rv = rws[slot]
        wsrc = ewv[slot]

        def _me(q, _):
            e = q * 4
            for u in range(4):
                wv = plsc.load_gather(wsrc, [jnp.full((16,), e + u, jnp.int32)])
                for j in range(CHUNK // 16):
                    rv[e + u, pl.ds(j * 16, 16)] = rv[e + u, pl.ds(j * 16, 16)] * wv
            return 0

        lax.fori_loop(0, EB1 // 4, _me, 0)

    _load_unpack(0, 0)
    _fire_gather(0)
    _load_unpack(1, 1)
    _fire_gather(1)

    def _steady(t, _):
        for u in range(3):
            b = 3 * t + u
            cur = u
            nxt = (u + 2) % 3

            _wait_gather(cur)
            _multiply(cur)

            @pl.when(b + 2 <= NB1 - 1)
            def _prefetch():
                def _do():
                    _wait_scatter(nxt)

                if u == 0:
                    pl.when(t > 0)(_do)
                else:
                    _do()
                _load_unpack(b + 2, nxt)
                _fire_gather(nxt)

            _fire_scatter(cur)

        return 0

    lax.fori_loop(0, NB1 // 3, _steady, 0)

    _wait_scatter(0)
    _wait_scatter(1)
    _wait_scatter(2)
    plsc.subcore_barrier()
    pltpu.sync_copy(acc.at[pl.ds(s * RPT, RPT)],
                    out_hbm.at[c, pl.ds(s * RPT, RPT)])


_stage_c = functools.partial(
    pl.kernel,
    out_type=jax.ShapeDtypeStruct((NC, NR, CHUNK), jnp.float32),
    mesh=_mesh,
    compiler_params=pltpu.CompilerParams(needs_layout_passes=False,
                                         use_tc_tiling_on_sc=False),
    scratch_types=[
        pltpu.VMEM((3 * EB1,), jnp.int32),
        pltpu.VMEM((3 * EB1,), jnp.int32),
        pltpu.VMEM((3 * EB1,), jnp.int32),
        pltpu.VMEM((EB1,), jnp.int32),
        pltpu.VMEM((EB1,), jnp.int32),
        pltpu.VMEM((EB1,), jnp.int32),
        pltpu.VMEM((EB1,), jnp.int32),
        pltpu.VMEM((EB1,), jnp.int32),
        pltpu.VMEM((EB1,), jnp.int32),
        pltpu.VMEM((EB1,), jnp.float32),
        pltpu.VMEM((EB1,), jnp.float32),
        pltpu.VMEM((EB1,), jnp.float32),
        pltpu.VMEM((EB1, CHUNK), jnp.float32),
        pltpu.VMEM((EB1, CHUNK), jnp.float32),
        pltpu.VMEM((EB1, CHUNK), jnp.float32),
        pltpu.VMEM_SHARED((NR, CHUNK), jnp.float32),
        pltpu.SemaphoreType.DMA,
        pltpu.SemaphoreType.DMA,
        pltpu.SemaphoreType.DMA,
        pltpu.SemaphoreType.DMA,
        pltpu.SemaphoreType.DMA,
        pltpu.SemaphoreType.DMA,
    ],
)(_agg1_body)


def _mm2_body(a_ref, ns_ref, nd_ref, w_ref, b_ref, o_ref):
    nd = nd_ref[...]
    ns = ns_ref[...]
    acc = None
    for ci in range(NC):
        h = jnp.maximum(a_ref[ci] * nd + b_ref[:, ci * CHUNK:(ci + 1) * CHUNK], 0.0)
        h = (h * ns).astype(jnp.bfloat16)
        w = w_ref[ci].astype(jnp.bfloat16)
        y = jnp.dot(h, w, preferred_element_type=jnp.float32)
        acc = y if acc is None else acc + y
    o_ref[...] = acc


def _stage_d(agg1, ns_col, nd_col, w2s, b1p):
    bn = 2048
    return pl.pallas_call(
        _mm2_body,
        grid=(NR // bn,),
        in_specs=[
            pl.BlockSpec((NC, bn, CHUNK), lambda i: (0, i, 0)),
            pl.BlockSpec((bn, 1), lambda i: (i, 0)),
            pl.BlockSpec((bn, 1), lambda i: (i, 0)),
            pl.BlockSpec((NC, CHUNK, C_P), lambda i: (0, 0, 0)),
            pl.BlockSpec((1, DH_P), lambda i: (0, 0)),
        ],
        out_specs=pl.BlockSpec((bn, C_P), lambda i: (i, 0)),
        out_shape=jax.ShapeDtypeStruct((NR, C_P), jnp.float32),
    )(agg1, ns_col, nd_col, w2s, b1p)


def _agg2_body(h_hbm, ei_hbm, ew_hbm, out_hbm,
               src_v, dst_v, ew_v, rows_v, acc, sem):
    c = lax.axis_index("c")
    s = lax.axis_index("s")

    zero16 = jnp.zeros((16,), jnp.float32)

    def _zr(i, _):
        for j in range(C_P // 16):
            rows_v[i, pl.ds(j * 16, 16)] = zero16
        return 0

    lax.fori_loop(0, EB2, _zr, 0)

    pltpu.sync_copy(rows_v.at[pl.ds(0, RPT)], acc.at[pl.ds(s * RPT, RPT)])
    plsc.subcore_barrier()

    ept2 = E // (NC * NS)
    base = c * (E // NC) + s * ept2

    def _blk(b, _):
        off = base + b * EB2
        pltpu.sync_copy(ei_hbm.at[pl.ds(off, EB2)], src_v)
        pltpu.sync_copy(ei_hbm.at[pl.ds(E + off, EB2)], dst_v)
        pltpu.sync_copy(ew_hbm.at[pl.ds(off, EB2)], ew_v)

        pltpu.async_copy(h_hbm.at[src_v], rows_v, sem).wait()

        def _me(e, _):
            wv = plsc.load_gather(ew_v, [jnp.full((16,), e, jnp.int32)])
            for j in range(C_P // 16):
                rows_v[e, pl.ds(j * 16, 16)] = rows_v[e, pl.ds(j * 16, 16)] * wv
            return 0

        lax.fori_loop(0, EB2, _me, 0)

        pltpu.sync_copy(rows_v, acc.at[dst_v], add=True)
        return 0

    lax.fori_loop(0, ept2 // EB2, _blk, 0)

    plsc.subcore_barrier()
    pltpu.sync_copy(acc.at[pl.ds(s * RPT, RPT)],
                    out_hbm.at[c, pl.ds(s * RPT, RPT)])


_stage_e = functools.partial(
    pl.kernel,
    out_type=jax.ShapeDtypeStruct((NC, NR, C_P), jnp.float32),
    mesh=_mesh,
    compiler_params=pltpu.CompilerParams(needs_layout_passes=False,
                                         use_tc_tiling_on_sc=False),
    scratch_types=[
        pltpu.VMEM((EB2,), jnp.int32),
        pltpu.VMEM((EB2,), jnp.int32),
        pltpu.VMEM((EB2,), jnp.float32),
        pltpu.VMEM((EB2, C_P), jnp.float32),
        pltpu.VMEM_SHARED((NR, C_P), jnp.float32),
        pltpu.SemaphoreType.DMA,
    ],
)(_agg2_body)


def _fin_body(p_ref, nd_ref, b_ref, o_ref):
    y = (p_ref[0] + p_ref[1]) * nd_ref[...] + b_ref[...]
    o_ref[...] = y[:, :40]


def _stage_f(parts, nd_col, b2p):
    bn = 2048
    return pl.pallas_call(
        _fin_body,
        grid=(NR // bn,),
        in_specs=[
            pl.BlockSpec((NC, bn, C_P), lambda i: (0, i, 0)),
            pl.BlockSpec((bn, 1), lambda i: (i, 0)),
            pl.BlockSpec((1, C_P), lambda i: (0, 0)),
        ],
        out_specs=pl.BlockSpec((bn, 40), lambda i: (i, 0)),
        out_shape=jax.ShapeDtypeStruct((NR, 40), jnp.float32),
    )(parts, nd_col, b2p)


def kernel(feature, edge_index, edge_weight, W1, b1, W2, b2):
    w1p = jnp.pad(W1, ((0, 0), (0, DH_P - W1.shape[1])))
    w1s = jnp.stack([w1p[:, i * CHUNK:(i + 1) * CHUNK] for i in range(NC)])
    w2p = jnp.pad(W2, ((0, DH_P - W2.shape[0]), (0, C_P - W2.shape[1])))
    w2s = jnp.stack([w2p[i * CHUNK:(i + 1) * CHUNK] for i in range(NC)])
    b1p = jnp.pad(b1, (0, DH_P - b1.shape[0])).reshape(1, DH_P)
    b2p = jnp.pad(b2, (0, C_P - b2.shape[0])).reshape(1, C_P)
    ei_flat = edge_index.reshape(2 * E)
    pk3 = jnp.stack([edge_index[0], edge_index[1],
                     lax.bitcast_convert_type(edge_weight, jnp.int32)],
                    axis=1).reshape(NS, E // NS, 3)
    fake = jnp.broadcast_to(jnp.array([[0, NR - 1, 0]], jnp.int32),
                            (NS, EPTP - E // NS, 3)).reshape(NS, EPTP - E // NS, 3)
    pk = jnp.concatenate([pk3, fake], axis=1).reshape(NS * EPTP * 3)

    nsd = _stage_b(ei_flat)
    ns_col = nsd[:NR].reshape(NR, 1)
    nd_col = nsd[NT:NT + NR].reshape(NR, 1)
    h1 = _stage_a(feature, w1s, ns_col)
    agg1 = _stage_c(h1, pk)
    h2 = _stage_d(agg1, ns_col, nd_col, w2s, b1p)
    parts = _stage_e(h2, ei_flat, edge_weight)
    return _stage_f(parts, nd_col, b2p)[:N]


# speedup vs baseline: 1.1448x; 1.0298x over previous
import functools

import jax
import jax.numpy as jnp
from jax import lax
from jax.experimental import pallas as pl
from jax.experimental.pallas import tpu as pltpu
from jax.experimental.pallas import tpu_sc as plsc

N = 10000
E = 320000
D_IN = 768
DH_P = 224
CHUNK = DH_P // 2
C_P = 48
NS = 16
NC = 2
NR = 10240
NT = 16384
EPT = E // NS
EB1 = 160
EB2 = 1000
RPT = NR // NS

_mesh = plsc.VectorSubcoreMesh(core_axis_name="c", subcore_axis_name="s", num_cores=2, num_subcores=16)


def _mm1_body(x_ref, w_ref, ns_ref, o_ref):
    x = x_ref[...].astype(jnp.bfloat16)
    w = w_ref[0].astype(jnp.bfloat16)
    o_ref[...] = jnp.dot(x, w, preferred_element_type=jnp.float32) * ns_ref[...]


def _stage_a(feature, w1s, ns_col):
    bn = 2048
    return pl.pallas_call(
        _mm1_body,
        grid=(NR // bn, NC),
        in_specs=[
            pl.BlockSpec((bn, D_IN), lambda i, c: (i, 0)),
            pl.BlockSpec((1, D_IN, CHUNK), lambda i, c: (c, 0, 0)),
            pl.BlockSpec((bn, 1), lambda i, c: (i, 0)),
        ],
        out_specs=pl.BlockSpec((bn, CHUNK), lambda i, c: (c * (NR // bn) + i, 0)),
        out_shape=jax.ShapeDtypeStruct((NC * NR, CHUNK), jnp.float32),
    )(feature, w1s, ns_col)


def _deg_body(ei_hbm, out_hbm, acc_v, ebuf, rbuf, dbuf, shared):
    c = lax.axis_index("c")
    s = lax.axis_index("s")
    zero16 = jnp.zeros((16,), jnp.float32)

    def _zacc(i, _):
        acc_v[pl.ds(i * 16, 16)] = zero16
        return 0

    lax.fori_loop(0, NT // 16, _zacc, 0)

    ones16 = jnp.full((16,), 1.0, jnp.float32)
    base = c * E + s * EPT

    def _blk(b, _):
        pltpu.sync_copy(ei_hbm.at[pl.ds(base + b * 2000, 2000)], ebuf)

        def _chunk(k, _):
            idx = ebuf[pl.ds(k * 16, 16)]
            plsc.addupdate_scatter(acc_v, [idx], ones16)
            return 0

        lax.fori_loop(0, 125, _chunk, 0)
        return 0

    lax.fori_loop(0, EPT // 2000, _blk, 0)

    pltpu.sync_copy(acc_v, shared.at[s])
    plsc.subcore_barrier()

    strip = s * (NT // NS)
    for r in range(NS):
        pltpu.sync_copy(shared.at[r, pl.ds(strip, NT // NS)], rbuf.at[r])

    def _red(k, _):
        tot = rbuf[0, pl.ds(k * 16, 16)]
        for r in range(1, NS):
            tot = tot + rbuf[r, pl.ds(k * 16, 16)]
        x = jnp.maximum(tot, 1.0)
        ii = 0x5F3759DF - lax.shift_right_logical(plsc.bitcast(x, jnp.int32), 1)
        y = plsc.bitcast(ii, jnp.float32)
        for _u in range(3):
            y = y * (1.5 - 0.5 * x * y * y)
        dbuf[pl.ds(k * 16, 16)] = y
        return 0

    lax.fori_loop(0, NT // NS // 16, _red, 0)
    pltpu.sync_copy(dbuf, out_hbm.at[pl.ds(c * NT + strip, NT // NS)])


_stage_b = functools.partial(
    pl.kernel,
    out_type=jax.ShapeDtypeStruct((2 * NT,), jnp.float32),
    mesh=_mesh,
    compiler_params=pltpu.CompilerParams(needs_layout_passes=False),
    scratch_types=[
        pltpu.VMEM((NT,), jnp.float32),
        pltpu.VMEM((2000,), jnp.int32),
        pltpu.VMEM((NS, NT // NS), jnp.float32),
        pltpu.VMEM((NT // NS,), jnp.float32),
        pltpu.VMEM_SHARED((NS, NT), jnp.float32),
    ],
)(_deg_body)


EPTP = 20160


def _make_agg(width, eb, eptp, split_edges):
    nb = eptp // eb
    assert nb % 3 == 0 and eb % 16 == 0 and eb % 4 == 0

    def body(h_hbm, pk_hbm, out_hbm,
             pk0, pk1, pk2, gi0, gi1, gi2, d0, d1, d2, ew0, ew1, ew2,
             r0, r1, r2, acc, g0, g1, g2, s0, s1, s2):
        c = lax.axis_index("c")
        s = lax.axis_index("s")
        pkv = (pk0, pk1, pk2)
        giv = (gi0, gi1, gi2)
        dsv = (d0, d1, d2)
        ewv = (ew0, ew1, ew2)
        rws = (r0, r1, r2)
        gsem = (g0, g1, g2)
        ssem = (s0, s1, s2)

        zero16 = jnp.zeros((16,), jnp.float32)

        def _zr(i, _):
            for j in range(width // 16):
                r0[i, pl.ds(j * 16, 16)] = zero16
            return 0

        lax.fori_loop(0, eb, _zr, 0)

        z = 0
        while z < RPT:
            n = min(eb, RPT - z)
            src = r0 if n == eb else r0.at[pl.ds(0, n)]
            pltpu.sync_copy(src, acc.at[pl.ds(s * RPT + z, n)])
            z += n
        plsc.subcore_barrier()

        if split_edges:
            base = (c * NS + s) * eptp
            goff = 0
        else:
            base = s * eptp
            goff = c * NR
        iota = lax.broadcasted_iota(jnp.int32, (16,), 0)

        def _load_unpack(b, slot):
            pltpu.sync_copy(pk_hbm.at[pl.ds(3 * (base + b * eb), 3 * eb)], pkv[slot])

            def _pc(k, _):
                r3 = (iota + k * 16) * 3
                giv[slot][pl.ds(k * 16, 16)] = plsc.load_gather(pkv[slot], [r3]) + goff
                dsv[slot][pl.ds(k * 16, 16)] = plsc.load_gather(pkv[slot], [r3 + 1])
                ewv[slot][pl.ds(k * 16, 16)] = plsc.bitcast(
                    plsc.load_gather(pkv[slot], [r3 + 2]), jnp.float32)
                return 0

            lax.fori_loop(0, eb // 16, _pc, 0)

        def _fire_gather(slot):
            return pltpu.async_copy(h_hbm.at[giv[slot]], rws[slot], gsem[slot])

        def _wait_gather(slot):
            pltpu.make_async_copy(h_hbm.at[giv[slot]], rws[slot], gsem[slot]).wait()

        def _fire_scatter(slot):
            pltpu.async_copy(rws[slot], acc.at[dsv[slot]], ssem[slot], add=True)

        def _wait_scatter(slot):
            pltpu.make_async_copy(rws[slot], acc.at[dsv[slot]], ssem[slot]).wait()

        def _multiply(slot):
            rv = rws[slot]
            wsrc = ewv[slot]

            def _me(q, _):
                e = q * 4
                for u in range(4):
                    wv = plsc.load_gather(wsrc, [jnp.full((16,), e + u, jnp.int32)])
                    for j in range(width // 16):
                        rv[e + u, pl.ds(j * 16, 16)] = rv[e + u, pl.ds(j * 16, 16)] * wv
                return 0

            lax.fori_loop(0, eb // 4, _me, 0)

        _load_unpack(0, 0)
        _fire_gather(0)
        _load_unpack(1, 1)
        _fire_gather(1)

        def _steady(t, _):
            for u in range(3):
                b = 3 * t + u
                nxt = (u + 2) % 3

                _wait_gather(u)
                _multiply(u)

                @pl.when(b + 2 <= nb - 1)
                def _prefetch():
                    def _do():
                        _wait_scatter(nxt)

                    if u == 0:
                        pl.when(t > 0)(_do)
                    else:
                        _do()
                    _load_unpack(b + 2, nxt)
                    _fire_gather(nxt)

                _fire_scatter(u)

            return 0

        lax.fori_loop(0, nb // 3, _steady, 0)

        _wait_scatter(0)
        _wait_scatter(1)
        _wait_scatter(2)
        plsc.subcore_barrier()
        pltpu.sync_copy(acc.at[pl.ds(s * RPT, RPT)],
                        out_hbm.at[c, pl.ds(s * RPT, RPT)])

    return functools.partial(
        pl.kernel,
        out_type=jax.ShapeDtypeStruct((NC, NR, width), jnp.float32),
        mesh=_mesh,
        compiler_params=pltpu.CompilerParams(needs_layout_passes=False,
                                             use_tc_tiling_on_sc=False),
        scratch_types=[
            pltpu.VMEM((3 * eb,), jnp.int32),
            pltpu.VMEM((3 * eb,), jnp.int32),
            pltpu.VMEM((3 * eb,), jnp.int32),
            pltpu.VMEM((eb,), jnp.int32),
            pltpu.VMEM((eb,), jnp.int32),
            pltpu.VMEM((eb,), jnp.int32),
            pltpu.VMEM((eb,), jnp.int32),
            pltpu.VMEM((eb,), jnp.int32),
            pltpu.VMEM((eb,), jnp.int32),
            pltpu.VMEM((eb,), jnp.float32),
            pltpu.VMEM((eb,), jnp.float32),
            pltpu.VMEM((eb,), jnp.float32),
            pltpu.VMEM((eb, width), jnp.float32),
            pltpu.VMEM((eb, width), jnp.float32),
            pltpu.VMEM((eb, width), jnp.float32),
            pltpu.VMEM_SHARED((NR, width), jnp.float32),
            pltpu.SemaphoreType.DMA,
            pltpu.SemaphoreType.DMA,
            pltpu.SemaphoreType.DMA,
            pltpu.SemaphoreType.DMA,
            pltpu.SemaphoreType.DMA,
            pltpu.SemaphoreType.DMA,
        ],
    )(body)


_stage_c = _make_agg(CHUNK, EB1, EPTP, split_edges=False)
_stage_e = _make_agg(C_P, 560, EPTP // 2, split_edges=True)


def _mm2_body(a_ref, ns_ref, nd_ref, w_ref, b_ref, o_ref):
    nd = nd_ref[...]
    ns = ns_ref[...]
    acc = None
    for ci in range(NC):
        h = jnp.maximum(a_ref[ci] * nd + b_ref[:, ci * CHUNK:(ci + 1) * CHUNK], 0.0)
        h = (h * ns).astype(jnp.bfloat16)
        w = w_ref[ci].astype(jnp.bfloat16)
        y = jnp.dot(h, w, preferred_element_type=jnp.float32)
        acc = y if acc is None else acc + y
    o_ref[...] = acc


def _stage_d(agg1, ns_col, nd_col, w2s, b1p):
    bn = 2048
    return pl.pallas_call(
        _mm2_body,
        grid=(NR // bn,),
        in_specs=[
            pl.BlockSpec((NC, bn, CHUNK), lambda i: (0, i, 0)),
            pl.BlockSpec((bn, 1), lambda i: (i, 0)),
            pl.BlockSpec((bn, 1), lambda i: (i, 0)),
            pl.BlockSpec((NC, CHUNK, C_P), lambda i: (0, 0, 0)),
            pl.BlockSpec((1, DH_P), lambda i: (0, 0)),
        ],
        out_specs=pl.BlockSpec((bn, C_P), lambda i: (i, 0)),
        out_shape=jax.ShapeDtypeStruct((NR, C_P), jnp.float32),
    )(agg1, ns_col, nd_col, w2s, b1p)


def _fin_body(p_ref, nd_ref, b_ref, o_ref):
    y = (p_ref[0] + p_ref[1]) * nd_ref[...] + b_ref[...]
    o_ref[...] = y[:, :40]


def _stage_f(parts, nd_col, b2p):
    bn = 2048
    return pl.pallas_call(
        _fin_body,
        grid=(NR // bn,),
        in_specs=[
            pl.BlockSpec((NC, bn, C_P), lambda i: (0, i, 0)),
            pl.BlockSpec((bn, 1), lambda i: (i, 0)),
            pl.BlockSpec((1, C_P), lambda i: (0, 0)),
        ],
        out_specs=pl.BlockSpec((bn, 40), lambda i: (i, 0)),
        out_shape=jax.ShapeDtypeStruct((NR, 40), jnp.float32),
    )(parts, nd_col, b2p)


def kernel(feature, edge_index, edge_weight, W1, b1, W2, b2):
    w1p = jnp.pad(W1, ((0, 0), (0, DH_P - W1.shape[1])))
    w1s = jnp.stack([w1p[:, i * CHUNK:(i + 1) * CHUNK] for i in range(NC)])
    w2p = jnp.pad(W2, ((0, DH_P - W2.shape[0]), (0, C_P - W2.shape[1])))
    w2s = jnp.stack([w2p[i * CHUNK:(i + 1) * CHUNK] for i in range(NC)])
    b1p = jnp.pad(b1, (0, DH_P - b1.shape[0])).reshape(1, DH_P)
    b2p = jnp.pad(b2, (0, C_P - b2.shape[0])).reshape(1, C_P)
    ei_flat = edge_index.reshape(2 * E)
    pk3 = jnp.stack([edge_index[0], edge_index[1],
                     lax.bitcast_convert_type(edge_weight, jnp.int32)],
                    axis=1).reshape(NS, E // NS, 3)
    fake = jnp.broadcast_to(jnp.array([[0, NR - 1, 0]], jnp.int32),
                            (NS, EPTP - E // NS, 3)).reshape(NS, EPTP - E // NS, 3)
    pk = jnp.concatenate([pk3, fake], axis=1).reshape(NS * EPTP * 3)

    nsd = _stage_b(ei_flat)
    ns_col = nsd[:NR].reshape(NR, 1)
    nd_col = nsd[NT:NT + NR].reshape(NR, 1)
    h1 = _stage_a(feature, w1s, ns_col)
    agg1 = _stage_c(h1, pk)
    h2 = _stage_d(agg1, ns_col, nd_col, w2s, b1p)
    parts = _stage_e(h2, pk)
    return _stage_f(parts, nd_col, b2p)[:N]
